# Initial kernel scaffold; baseline (speedup 1.0000x reference)
import functools

import jax
import jax.numpy as jnp
from jax import lax
from jax.experimental import pallas as pl
from jax.experimental.pallas import tpu as pltpu
from jax.experimental.pallas import tpu_sc as plsc

B = 16384
USER_DIM = 64
GENRE_DIM = 32
GPI = 10
NC, NS, L = 2, 16, 16
NW = NC * NS
BPW = B // NW
CH = 128
NCH = BPW // CH


def _body(uid_hbm, gid_hbm, utab_hbm, gtab_hbm, out_hbm,
          uidx_v, gidx_v, urows_v, tmp_v, gout_v, usem, gsem):
  cid = lax.axis_index("c")
  sid = lax.axis_index("s")
  wid = sid * NC + cid
  base = wid * BPW

  pltpu.sync_copy(uid_hbm.at[wid], uidx_v)
  pltpu.sync_copy(gid_hbm.at[wid], gidx_v)

  for c in range(NCH):
    for t in range(CH // L):
      uidx_v[c, pl.ds(t * L, L)] = uidx_v[c, pl.ds(t * L, L)] + 1

  ucopies = [
      pltpu.async_copy(utab_hbm.at[uidx_v.at[c]],
                       urows_v.at[pl.ds(c * CH, CH)], usem)
      for c in range(NCH)
  ]

  for c in range(NCH):
    gcopies = [
        pltpu.async_copy(gtab_hbm.at[gidx_v.at[GPI * c + k]],
                         tmp_v.at[pl.ds(k * CH, CH)], gsem)
        for k in range(GPI)
    ]
    for cp in gcopies:
      cp.wait()

    def item(ii, carry, c=c):
      row0 = ii * GPI
      a0 = tmp_v[row0, pl.ds(0, L)]
      a1 = tmp_v[row0, pl.ds(L, L)]
      for j in range(1, GPI):
        a0 = a0 + tmp_v[row0 + j, pl.ds(0, L)]
        a1 = a1 + tmp_v[row0 + j, pl.ds(L, L)]
      scale = jnp.float32(0.1)
      gout_v[c * CH + ii, pl.ds(0, L)] = a0 * scale
      gout_v[c * CH + ii, pl.ds(L, L)] = a1 * scale
      return carry

    lax.fori_loop(0, CH, item, 0)

  for cp in ucopies:
    cp.wait()

  pltpu.sync_copy(urows_v, out_hbm.at[pl.ds(base, BPW), pl.ds(0, USER_DIM)])
  pltpu.sync_copy(gout_v,
                  out_hbm.at[pl.ds(base, BPW), pl.ds(USER_DIM, GENRE_DIM)])


@jax.jit
def kernel(user_id, movie_genres, user_table, genre_table):
  uid3 = user_id.reshape(NW, NCH, CH)
  gid3 = movie_genres.reshape(NW, GPI * NCH, CH)

  run = pl.kernel(
      _body,
      out_type=jax.ShapeDtypeStruct((B, USER_DIM + GENRE_DIM), jnp.float32),
      mesh=plsc.VectorSubcoreMesh(core_axis_name="c", subcore_axis_name="s",
                                  num_cores=NC, num_subcores=NS),
      scratch_types=[
          pltpu.VMEM((NCH, CH), jnp.int32),
          pltpu.VMEM((GPI * NCH, CH), jnp.int32),
          pltpu.VMEM((BPW, USER_DIM), jnp.float32),
          pltpu.VMEM((GPI * CH, GENRE_DIM), jnp.float32),
          pltpu.VMEM((BPW, GENRE_DIM), jnp.float32),
          pltpu.SemaphoreType.DMA,
          pltpu.SemaphoreType.DMA,
      ],
  )
  return run(uid3, gid3, user_table, genre_table)


# speedup vs baseline: 2.2573x; 2.2573x over previous
import functools

import jax
import jax.numpy as jnp
from jax import lax
from jax.experimental import pallas as pl
from jax.experimental.pallas import tpu as pltpu
from jax.experimental.pallas import tpu_sc as plsc

B = 16384
USER_DIM = 64
GENRE_DIM = 32
GPI = 10
NC, NS, L = 2, 16, 16
NW = NC * NS
BPW = B // NW
CH = 128
NCH = BPW // CH


def _body(uid_hbm, gid_hbm, utab_hbm, gtab_hbm, out_hbm,
          uidx_v, gidx_v, urows_v, gout_v, tmp_v, usem, gsem):
  cid = lax.axis_index("c")
  sid = lax.axis_index("s")
  wid = sid * NC + cid
  base = wid * BPW

  pltpu.sync_copy(uid_hbm.at[wid], uidx_v)
  pltpu.sync_copy(gid_hbm.at[wid], gidx_v)

  for c in range(NCH):
    for t in range(CH // L):
      uidx_v[c, pl.ds(t * L, L)] = uidx_v[c, pl.ds(t * L, L)] + 1

  ucopies = [
      pltpu.async_copy(utab_hbm.at[uidx_v.at[c]],
                       urows_v.at[pl.ds(c * CH, CH)], usem)
      for c in range(NCH)
  ]

  for c in range(NCH):
    gcopies = [
        pltpu.async_copy(gtab_hbm.at[gidx_v.at[GPI * c + k]],
                         tmp_v.at[pl.ds(k * CH, CH)], gsem)
        for k in range(GPI)
    ]
    for cp in gcopies:
      cp.wait()

    def item(ii, carry, c=c):
      row0 = ii * GPI
      a0 = tmp_v[row0, pl.ds(0, L)]
      a1 = tmp_v[row0, pl.ds(L, L)]
      for j in range(1, GPI):
        a0 = a0 + tmp_v[row0 + j, pl.ds(0, L)]
        a1 = a1 + tmp_v[row0 + j, pl.ds(L, L)]
      scale = jnp.float32(0.1)
      gout_v[c * CH + ii, pl.ds(0, L)] = a0 * scale
      gout_v[c * CH + ii, pl.ds(L, L)] = a1 * scale
      return carry

    lax.fori_loop(0, CH, item, 0)

  for cp in ucopies:
    cp.wait()

  pltpu.sync_copy(urows_v, out_hbm.at[pl.ds(base, BPW), pl.ds(0, USER_DIM)])
  pltpu.sync_copy(gout_v,
                  out_hbm.at[pl.ds(base, BPW), pl.ds(USER_DIM, GENRE_DIM)])


@jax.jit
def kernel(user_id, movie_genres, user_table, genre_table):
  uid3 = user_id.reshape(NW, NCH, CH)
  gid3 = movie_genres.reshape(NW, GPI * NCH, CH)

  run = pl.kernel(
      _body,
      out_type=jax.ShapeDtypeStruct((B, USER_DIM + GENRE_DIM), jnp.float32),
      mesh=plsc.VectorSubcoreMesh(core_axis_name="c", subcore_axis_name="s",
                                  num_cores=NC, num_subcores=NS),
      scratch_types=[
          pltpu.VMEM((NCH, CH), jnp.int32),
          pltpu.VMEM((GPI * NCH, CH), jnp.int32),
          pltpu.VMEM((BPW, USER_DIM), jnp.float32),
          pltpu.VMEM((BPW, GENRE_DIM), jnp.float32),
          pltpu.VMEM((GPI * CH, GENRE_DIM), jnp.float32),
          pltpu.SemaphoreType.DMA,
          pltpu.SemaphoreType.DMA,
      ],
      compiler_params=pltpu.CompilerParams(use_tc_tiling_on_sc=False),
  )
  return run(uid3, gid3, user_table, genre_table)


# speedup vs baseline: 2.2719x; 1.0065x over previous
import functools

import jax
import jax.numpy as jnp
from jax import lax
from jax.experimental import pallas as pl
from jax.experimental.pallas import tpu as pltpu
from jax.experimental.pallas import tpu_sc as plsc

B = 16384
USER_DIM = 64
GENRE_DIM = 32
GPI = 10
NC, NS, L = 2, 16, 16
NW = NC * NS
BPW = B // NW
CH = 128
NCH = BPW // CH


def _body(uid_hbm, gid_hbm, utab_hbm, gtab_hbm, out_hbm,
          uidx_v, gidx_v, urows_v, gband_v, tmp_v,
          usem, gsemA, gsemB, wsemA, wsemB):
  cid = lax.axis_index("c")
  sid = lax.axis_index("s")
  wid = sid * NC + cid
  base = wid * BPW
  gsems = (gsemA, gsemB)
  wsems = (wsemA, wsemB)

  pltpu.sync_copy(uid_hbm.at[wid], uidx_v)
  pltpu.sync_copy(gid_hbm.at[wid], gidx_v)

  for c in range(NCH):
    for t in range(CH // L):
      uidx_v[c, pl.ds(t * L, L)] = uidx_v[c, pl.ds(t * L, L)] + 1

  ucopies = [
      pltpu.async_copy(utab_hbm.at[uidx_v.at[c]],
                       urows_v.at[pl.ds(c * CH, CH)], usem)
      for c in range(NCH)
  ]

  def fire_chunk(c):
    return [
        pltpu.async_copy(gtab_hbm.at[gidx_v.at[GPI * c + k]],
                         tmp_v.at[c % 2, pl.ds(k * CH, CH)], gsems[c % 2])
        for k in range(GPI)
    ]

  pend = fire_chunk(0)
  wcopies = []
  for c in range(NCH):
    nxt = fire_chunk(c + 1) if c + 1 < NCH else []
    for cp in pend:
      cp.wait()
    pend = nxt
    tb = c % 2

    if len(wcopies) >= 2:
      wcopies[c - 2].wait()

    scale = jnp.float32(0.1)

    @functools.partial(plsc.parallel_loop, 0, CH, unroll=8)
    def _item(i, tb=tb):
      row0 = i * GPI
      for h in range(2):
        sl = pl.ds(h * L, L)
        t = [tmp_v[tb, row0 + j, sl] for j in range(GPI)]
        s = (((t[0] + t[1]) + (t[2] + t[3])) +
             ((t[4] + t[5]) + (t[6] + t[7])) + (t[8] + t[9]))
        gband_v[tb, i, sl] = s * scale

    wcopies.append(
        pltpu.async_copy(
            gband_v.at[tb],
            out_hbm.at[pl.ds(base + c * CH, CH),
                       pl.ds(USER_DIM, GENRE_DIM)], wsems[tb]))

  for cp in ucopies:
    cp.wait()

  pltpu.sync_copy(urows_v, out_hbm.at[pl.ds(base, BPW), pl.ds(0, USER_DIM)])
  for cp in wcopies[-2:]:
    cp.wait()


@jax.jit
def kernel(user_id, movie_genres, user_table, genre_table):
  uid3 = user_id.reshape(NW, NCH, CH)
  gid3 = movie_genres.reshape(NW, GPI * NCH, CH)

  run = pl.kernel(
      _body,
      out_type=jax.ShapeDtypeStruct((B, USER_DIM + GENRE_DIM), jnp.float32),
      mesh=plsc.VectorSubcoreMesh(core_axis_name="c", subcore_axis_name="s",
                                  num_cores=NC, num_subcores=NS),
      scratch_types=[
          pltpu.VMEM((NCH, CH), jnp.int32),
          pltpu.VMEM((GPI * NCH, CH), jnp.int32),
          pltpu.VMEM((BPW, USER_DIM), jnp.float32),
          pltpu.VMEM((2, CH, GENRE_DIM), jnp.float32),
          pltpu.VMEM((2, GPI * CH, GENRE_DIM), jnp.float32),
          pltpu.SemaphoreType.DMA,
          pltpu.SemaphoreType.DMA,
          pltpu.SemaphoreType.DMA,
          pltpu.SemaphoreType.DMA,
          pltpu.SemaphoreType.DMA,
      ],
      compiler_params=pltpu.CompilerParams(use_tc_tiling_on_sc=False),
  )
  return run(uid3, gid3, user_table, genre_table)


# speedup vs baseline: 5.6719x; 2.4965x over previous
import functools

import jax
import jax.numpy as jnp
from jax import lax
from jax.experimental import pallas as pl
from jax.experimental.pallas import tpu as pltpu
from jax.experimental.pallas import tpu_sc as plsc

B = 16384
USER_DIM = 64
GENRE_DIM = 32
GVOC = 21
GPI = 10
NC, NS, L = 2, 16, 16
NW = NC * NS
BPW = B // NW
CH = 128
NCH = BPW // CH
NG = BPW // L


def _body(uid_hbm, gidT_hbm, utab_hbm, gtab_hbm, out_hbm,
          uidx_v, gidxT_v, gtab_v, urows_v, gout_v, usem):
  cid = lax.axis_index("c")
  sid = lax.axis_index("s")
  wid = sid * NC + cid
  base = wid * BPW

  pltpu.sync_copy(uid_hbm.at[wid], uidx_v)
  pltpu.sync_copy(gidT_hbm.at[wid], gidxT_v)
  pltpu.sync_copy(gtab_hbm, gtab_v)

  for c in range(NCH):
    for t in range(CH // L):
      uidx_v[c, pl.ds(t * L, L)] = uidx_v[c, pl.ds(t * L, L)] + 1

  ucopies = [
      pltpu.async_copy(utab_hbm.at[uidx_v.at[c]],
                       urows_v.at[pl.ds(c * CH, CH)], usem)
      for c in range(NCH)
  ]

  scale = jnp.float32(1.0 / GPI)
  iota = lax.iota(jnp.int32, L)

  def _grp(t, carry):
    rowsel = [gidxT_v[j, pl.ds(t * L, L)] for j in range(GPI)]
    item_rows = iota + t * L
    for c in range(GENRE_DIM):
      cvec = jnp.full((L,), c, jnp.int32)
      acc = plsc.load_gather(gtab_v, [rowsel[0], cvec])
      for j in range(1, GPI):
        acc = acc + plsc.load_gather(gtab_v, [rowsel[j], cvec])
      plsc.store_scatter(gout_v, [item_rows, cvec], acc * scale)
    return carry

  lax.fori_loop(0, NG, _grp, 0)

  for cp in ucopies:
    cp.wait()

  pltpu.sync_copy(urows_v, out_hbm.at[pl.ds(base, BPW), pl.ds(0, USER_DIM)])
  pltpu.sync_copy(gout_v,
                  out_hbm.at[pl.ds(base, BPW), pl.ds(USER_DIM, GENRE_DIM)])


@jax.jit
def kernel(user_id, movie_genres, user_table, genre_table):
  uid3 = user_id.reshape(NW, NCH, CH)
  gidT = jnp.transpose(movie_genres.reshape(NW, BPW, GPI), (0, 2, 1))

  run = pl.kernel(
      _body,
      out_type=jax.ShapeDtypeStruct((B, USER_DIM + GENRE_DIM), jnp.float32),
      mesh=plsc.VectorSubcoreMesh(core_axis_name="c", subcore_axis_name="s",
                                  num_cores=NC, num_subcores=NS),
      scratch_types=[
          pltpu.VMEM((NCH, CH), jnp.int32),
          pltpu.VMEM((GPI, BPW), jnp.int32),
          pltpu.VMEM((GVOC, GENRE_DIM), jnp.float32),
          pltpu.VMEM((BPW, USER_DIM), jnp.float32),
          pltpu.VMEM((BPW, GENRE_DIM), jnp.float32),
          pltpu.SemaphoreType.DMA,
      ],
      compiler_params=pltpu.CompilerParams(use_tc_tiling_on_sc=False,
                                           needs_layout_passes=False),
  )
  return run(uid3, gidT, user_table, genre_table)
